# baseline (speedup 1.0000x reference)
import jax
import jax.numpy as jnp
from jax.experimental import pallas as pl
from jax.experimental.pallas import tpu as pltpu

_INV_STEPS = 1.0 / 16.0
_BLOCK_ROWS = 256


def _body(i_ref, d_ref, nd_ref, o_ref):
    s = pl.program_id(1)
    hit = s == i_ref[0]

    @pl.when(hit)
    def _():
        o_ref[...] = d_ref[...] + nd_ref[...][None] * _INV_STEPS

    @pl.when(jnp.logical_not(hit))
    def _():
        o_ref[...] = d_ref[...]


def kernel(data, new_data, i):
    steps, rows, cols = data.shape
    i_arr = jnp.asarray(i, jnp.int32).reshape((1,))
    grid = (rows // _BLOCK_ROWS, steps)
    return pl.pallas_call(
        _body,
        grid=grid,
        in_specs=[
            pl.BlockSpec(memory_space=pltpu.SMEM),
            pl.BlockSpec((1, _BLOCK_ROWS, cols), lambda r, s: (s, r, 0)),
            pl.BlockSpec((_BLOCK_ROWS, cols), lambda r, s: (r, 0)),
        ],
        out_specs=pl.BlockSpec((1, _BLOCK_ROWS, cols), lambda r, s: (s, r, 0)),
        out_shape=jax.ShapeDtypeStruct(data.shape, data.dtype),
        compiler_params=pltpu.CompilerParams(
            dimension_semantics=("arbitrary", "arbitrary"),
        ),
    )(i_arr, data, new_data)


# speedup vs baseline: 2.3797x; 2.3797x over previous
import jax
import jax.numpy as jnp
from jax.experimental import pallas as pl
from jax.experimental.pallas import tpu as pltpu

_INV_STEPS = 1.0 / 16.0
_BLOCK_ROWS = 1024


def _body(i_ref, d_ref, nd_ref, o_ref):
    s = pl.program_id(1)
    hit = s == i_ref[0]

    @pl.when(hit)
    def _():
        o_ref[...] = d_ref[...] + nd_ref[...][None] * _INV_STEPS

    @pl.when(jnp.logical_not(hit))
    def _():
        o_ref[...] = d_ref[...]


def kernel(data, new_data, i):
    steps, rows, cols = data.shape
    i_arr = jnp.asarray(i, jnp.int32).reshape((1,))
    grid = (rows // _BLOCK_ROWS, steps)
    return pl.pallas_call(
        _body,
        grid=grid,
        in_specs=[
            pl.BlockSpec(memory_space=pltpu.SMEM),
            pl.BlockSpec((1, _BLOCK_ROWS, cols), lambda r, s: (s, r, 0)),
            pl.BlockSpec((_BLOCK_ROWS, cols), lambda r, s: (r, 0)),
        ],
        out_specs=pl.BlockSpec((1, _BLOCK_ROWS, cols), lambda r, s: (s, r, 0)),
        out_shape=jax.ShapeDtypeStruct(data.shape, data.dtype),
        compiler_params=pltpu.CompilerParams(
            dimension_semantics=("arbitrary", "arbitrary"),
        ),
    )(i_arr, data, new_data)


# speedup vs baseline: 3.2806x; 1.3786x over previous
import jax
import jax.numpy as jnp
from jax.experimental import pallas as pl
from jax.experimental.pallas import tpu as pltpu

_INV_STEPS = 1.0 / 16.0
_BLOCK_ROWS = 2048


def _body(i_ref, d_ref, nd_ref, o_ref):
    s = pl.program_id(1)
    hit = s == i_ref[0]

    @pl.when(hit)
    def _():
        o_ref[...] = d_ref[...] + nd_ref[...][None] * _INV_STEPS

    @pl.when(jnp.logical_not(hit))
    def _():
        o_ref[...] = d_ref[...]


def kernel(data, new_data, i):
    steps, rows, cols = data.shape
    i_arr = jnp.asarray(i, jnp.int32).reshape((1,))
    grid = (rows // _BLOCK_ROWS, steps)
    return pl.pallas_call(
        _body,
        grid=grid,
        in_specs=[
            pl.BlockSpec(memory_space=pltpu.SMEM),
            pl.BlockSpec((1, _BLOCK_ROWS, cols), lambda r, s: (s, r, 0)),
            pl.BlockSpec((_BLOCK_ROWS, cols), lambda r, s: (r, 0)),
        ],
        out_specs=pl.BlockSpec((1, _BLOCK_ROWS, cols), lambda r, s: (s, r, 0)),
        out_shape=jax.ShapeDtypeStruct(data.shape, data.dtype),
        compiler_params=pltpu.CompilerParams(
            dimension_semantics=("arbitrary", "arbitrary"),
        ),
    )(i_arr, data, new_data)


# speedup vs baseline: 3.5862x; 1.0931x over previous
import jax
import jax.numpy as jnp
from jax.experimental import pallas as pl
from jax.experimental.pallas import tpu as pltpu

_INV_STEPS = 1.0 / 16.0
_BLOCK_ROWS = 4096


def _body(i_ref, d_ref, nd_ref, o_ref):
    s = pl.program_id(1)
    hit = s == i_ref[0]

    @pl.when(hit)
    def _():
        o_ref[...] = d_ref[...] + nd_ref[...][None] * _INV_STEPS

    @pl.when(jnp.logical_not(hit))
    def _():
        o_ref[...] = d_ref[...]


def kernel(data, new_data, i):
    steps, rows, cols = data.shape
    i_arr = jnp.asarray(i, jnp.int32).reshape((1,))
    grid = (rows // _BLOCK_ROWS, steps)
    return pl.pallas_call(
        _body,
        grid=grid,
        in_specs=[
            pl.BlockSpec(memory_space=pltpu.SMEM),
            pl.BlockSpec((1, _BLOCK_ROWS, cols), lambda r, s: (s, r, 0)),
            pl.BlockSpec((_BLOCK_ROWS, cols), lambda r, s: (r, 0)),
        ],
        out_specs=pl.BlockSpec((1, _BLOCK_ROWS, cols), lambda r, s: (s, r, 0)),
        out_shape=jax.ShapeDtypeStruct(data.shape, data.dtype),
        compiler_params=pltpu.CompilerParams(
            dimension_semantics=("arbitrary", "arbitrary"),
        ),
    )(i_arr, data, new_data)


# speedup vs baseline: 3.6758x; 1.0250x over previous
import jax
import jax.numpy as jnp
from jax.experimental import pallas as pl
from jax.experimental.pallas import tpu as pltpu

_INV_STEPS = 1.0 / 16.0
_BLOCK_ROWS = 8192


def _body(i_ref, d_ref, nd_ref, o_ref):
    s = pl.program_id(1)
    hit = s == i_ref[0]

    @pl.when(hit)
    def _():
        o_ref[...] = d_ref[...] + nd_ref[...][None] * _INV_STEPS

    @pl.when(jnp.logical_not(hit))
    def _():
        o_ref[...] = d_ref[...]


def kernel(data, new_data, i):
    steps, rows, cols = data.shape
    i_arr = jnp.asarray(i, jnp.int32).reshape((1,))
    grid = (rows // _BLOCK_ROWS, steps)
    return pl.pallas_call(
        _body,
        grid=grid,
        in_specs=[
            pl.BlockSpec(memory_space=pltpu.SMEM),
            pl.BlockSpec((1, _BLOCK_ROWS, cols), lambda r, s: (s, r, 0)),
            pl.BlockSpec((_BLOCK_ROWS, cols), lambda r, s: (r, 0)),
        ],
        out_specs=pl.BlockSpec((1, _BLOCK_ROWS, cols), lambda r, s: (s, r, 0)),
        out_shape=jax.ShapeDtypeStruct(data.shape, data.dtype),
        compiler_params=pltpu.CompilerParams(
            dimension_semantics=("arbitrary", "arbitrary"),
        ),
    )(i_arr, data, new_data)
